# baseline (speedup 1.0000x reference)
import functools

import jax
import jax.numpy as jnp
from jax import lax
from jax.experimental import pallas as pl
from jax.experimental.pallas import tpu as pltpu
from jax.experimental.pallas import tpu_sc as plsc

D = 128
N_CAT = 26
D_CONT = D - N_CAT
VOCAB = 1000
EMB_DIM = 32
BATCH = 4096
EMB_W = N_CAT * EMB_DIM

NUM_CORES = 2
NUM_SUBCORES = 16
NW = NUM_CORES * NUM_SUBCORES
ROWS = BATCH // NW
CHUNK = 64
LANES = 16

CAT_BASE = 96
CAT_OFF = D_CONT - CAT_BASE


def _body(x_hbm, tab_hbm, emb_hbm, gath, ebuf, xcat, idx2d, sem_g):
    wid = lax.axis_index("s") * NUM_CORES + lax.axis_index("c")
    lanes = lax.iota(jnp.int32, LANES)

    def chunk_pass(c, carry):
        base = wid * ROWS + c * CHUNK

        pltpu.sync_copy(
            x_hbm.at[pl.ds(base, CHUNK), pl.ds(CAT_BASE, 32)], xcat
        )

        def feat(i, cc):
            col = jnp.full((LANES,), i + CAT_OFF, jnp.int32)
            for m in range(CHUNK // LANES):
                rows = m * LANES + lanes
                v = plsc.load_gather(xcat, [rows, col])
                idx2d[i, pl.ds(m * LANES, LANES)] = v.astype(jnp.int32)
            pltpu.make_async_copy(
                tab_hbm.at[i].at[idx2d.at[i]], gath.at[i], sem_g
            ).start()
            return cc

        lax.fori_loop(0, N_CAT, feat, 0)

        def drain(g, cc):
            pltpu.make_async_copy(
                tab_hbm.at[g].at[idx2d.at[g]], gath.at[g], sem_g
            ).wait()
            return cc

        lax.fori_loop(0, N_CAT, drain, 0)

        def place_row(r, cc):
            rb = r * EMB_W
            for g in range(N_CAT):
                dst = rb + g * EMB_DIM
                lo = gath[g, r, pl.ds(0, LANES)]
                hi = gath[g, r, pl.ds(LANES, LANES)]
                ebuf[pl.ds(dst, LANES)] = lo
                ebuf[pl.ds(dst + LANES, LANES)] = hi
            return cc

        lax.fori_loop(0, CHUNK, place_row, 0)

        pltpu.sync_copy(
            ebuf, emb_hbm.at[pl.ds(base * EMB_W, CHUNK * EMB_W)]
        )
        return carry

    lax.fori_loop(0, ROWS // CHUNK, chunk_pass, 0)


_sc_call = functools.partial(
    pl.kernel,
    mesh=plsc.VectorSubcoreMesh(core_axis_name="c", subcore_axis_name="s"),
    out_type=jax.ShapeDtypeStruct((BATCH * EMB_W,), jnp.float32),
    scratch_types=[
        pltpu.VMEM((N_CAT, CHUNK, EMB_DIM), jnp.float32),
        pltpu.VMEM((CHUNK * EMB_W,), jnp.float32),
        pltpu.VMEM((CHUNK, 32), jnp.float32),
        pltpu.VMEM((N_CAT, CHUNK), jnp.int32),
        pltpu.SemaphoreType.DMA,
    ],
    compiler_params=pltpu.CompilerParams(
        use_tc_tiling_on_sc=False, needs_layout_passes=False
    ),
)(_body)


@jax.jit
def kernel(X, emb_tables, categ_idcs, non_categ_mask):
    emb = _sc_call(X, emb_tables).reshape(BATCH, EMB_W)
    return jnp.concatenate([X[:, :D_CONT], emb], axis=-1)


# speedup vs baseline: 1.1557x; 1.1557x over previous
import functools

import jax
import jax.numpy as jnp
from jax import lax
from jax.experimental import pallas as pl
from jax.experimental.pallas import tpu as pltpu
from jax.experimental.pallas import tpu_sc as plsc

D = 128
N_CAT = 26
D_CONT = D - N_CAT
VOCAB = 1000
EMB_DIM = 32
BATCH = 4096
EMB_W = N_CAT * EMB_DIM

NUM_CORES = 2
NUM_SUBCORES = 16
NW = NUM_CORES * NUM_SUBCORES
ROWS = BATCH // NW
LANES = 16

CAT_BASE = 96
CAT_OFF = D_CONT - CAT_BASE


def _body(x_hbm, tab_hbm, emb_hbm, gath, xcat, idx2d, sem_g, sem_w):
    wid = lax.axis_index("s") * NUM_CORES + lax.axis_index("c")
    base = wid * ROWS
    lanes = lax.iota(jnp.int32, LANES)

    pltpu.sync_copy(x_hbm.at[pl.ds(base, ROWS), pl.ds(CAT_BASE, 32)], xcat)

    def feat(i, cc):
        col = jnp.full((LANES,), i + CAT_OFF, jnp.int32)
        for m in range(ROWS // LANES):
            rows = m * LANES + lanes
            v = plsc.load_gather(xcat, [rows, col])
            idx2d[i, pl.ds(m * LANES, LANES)] = v.astype(jnp.int32)
        pltpu.make_async_copy(
            tab_hbm.at[i].at[idx2d.at[i]], gath.at[i], sem_g.at[i]
        ).start()
        return cc

    lax.fori_loop(0, N_CAT, feat, 0)

    def pipe(g, cc):
        pltpu.make_async_copy(
            tab_hbm.at[g].at[idx2d.at[g]], gath.at[g], sem_g.at[g]
        ).wait()
        pltpu.make_async_copy(
            gath.at[g],
            emb_hbm.at[pl.ds(base, ROWS), pl.ds(g * EMB_DIM, EMB_DIM)],
            sem_w,
        ).start()
        return cc

    lax.fori_loop(0, N_CAT, pipe, 0)

    def drain_out(g, cc):
        pltpu.make_async_copy(
            gath.at[g],
            emb_hbm.at[pl.ds(base, ROWS), pl.ds(g * EMB_DIM, EMB_DIM)],
            sem_w,
        ).wait()
        return cc

    lax.fori_loop(0, N_CAT, drain_out, 0)


_sc_call = functools.partial(
    pl.kernel,
    mesh=plsc.VectorSubcoreMesh(core_axis_name="c", subcore_axis_name="s"),
    out_type=jax.ShapeDtypeStruct((BATCH, EMB_W), jnp.float32),
    scratch_types=[
        pltpu.VMEM((N_CAT, ROWS, EMB_DIM), jnp.float32),
        pltpu.VMEM((ROWS, 32), jnp.float32),
        pltpu.VMEM((N_CAT, ROWS), jnp.int32),
        pltpu.SemaphoreType.DMA((N_CAT,)),
        pltpu.SemaphoreType.DMA,
    ],
    compiler_params=pltpu.CompilerParams(
        use_tc_tiling_on_sc=False, needs_layout_passes=False
    ),
)(_body)


@jax.jit
def kernel(X, emb_tables, categ_idcs, non_categ_mask):
    emb = _sc_call(X, emb_tables)
    return jnp.concatenate([X[:, :D_CONT], emb], axis=-1)
